# baseline (speedup 1.0000x reference)
import functools

import jax
import jax.numpy as jnp
from jax import lax
from jax.experimental import pallas as pl
from jax.experimental.pallas import tpu as pltpu
from jax.experimental.pallas import tpu_sc as plsc

NC, NS = 2, 16
NW = NC * NS
N = 10000
E = 320000
G = 16
EPS = 1e-5

CH = 512
EP = 327680
EPT = EP // NS
CPT = EPT // CH
CHD = 128
EPT_D = EP // NW
CPT_D = EPT_D // CHD
N_ACC = 10240
DUMMY = N
ZPT = N_ACC // NS
RPT = 640
RPT_LAST = N - RPT * (NS - 1)
HW = 32

_MESH = plsc.VectorSubcoreMesh(core_axis_name="c", subcore_axis_name="s",
                               num_cores=NC, num_subcores=NS)


def _make_edge_pass():
    @functools.partial(
        pl.kernel,
        out_type=jax.ShapeDtypeStruct((NC, N, HW), jnp.float32),
        mesh=_MESH,
        scratch_types=[
            pltpu.VMEM((CPT, CH), jnp.int32),
            pltpu.VMEM((CPT, CH), jnp.int32),
            pltpu.VMEM((CH, HW), jnp.float32),
            pltpu.VMEM((CH, HW), jnp.float32),
            pltpu.VMEM((CH, HW), jnp.float32),
            pltpu.VMEM((CH, HW), jnp.float32),
            pltpu.VMEM_SHARED((N_ACC, HW), jnp.float32),
            pltpu.SemaphoreType.DMA,
            pltpu.SemaphoreType.DMA,
        ],
        compiler_params=pltpu.CompilerParams(use_tc_tiling_on_sc=False),
    )
    def k(featA, featB, srcr, dstr, out, sidx, didx, rows0, rows1, rows2,
          rows3, acc, gsem, ssem):
        cid = lax.axis_index("c")
        sid = lax.axis_index("s")
        rows = (rows0, rows1, rows2, rows3)
        pltpu.sync_copy(srcr.at[sid], sidx)
        pltpu.sync_copy(dstr.at[sid], didx)

        def zrow(i, _):
            for j in range(HW // 16):
                rows0[i, pl.ds(j * 16, 16)] = jnp.zeros((16,), jnp.float32)
            return 0
        lax.fori_loop(0, CH, zrow, 0)
        zbase = sid * ZPT
        def zcp(i, _):
            pltpu.sync_copy(rows0.at[pl.ds(0, ZPT // 2)],
                            acc.at[pl.ds(zbase + i * (ZPT // 2), ZPT // 2)])
            return 0
        lax.fori_loop(0, 2, zcp, 0)
        plsc.subcore_barrier()

        def run(feat):
            GS = 2
            NG = CPT // GS
            A = (rows0, rows1)
            B = (rows2, rows3)

            def gstart(j, buf):
                pltpu.async_copy(feat.at[sidx.at[j]], buf, gsem)

            def gwait():
                pltpu.make_async_copy(feat.at[sidx.at[0]], rows0, gsem).wait()

            def sstart(j, buf):
                pltpu.async_copy(buf, acc.at[didx.at[j]], ssem, add=True)

            def swait():
                pltpu.make_async_copy(feat.at[pl.ds(0, CH)], rows0,
                                      ssem).wait()

            gstart(0, A[0])
            gstart(1, A[1])

            def group(g, cur, oth):
                gwait(); gwait()
                sstart(g * GS, cur[0])
                sstart(g * GS + 1, cur[1])
                @pl.when(g > 0)
                def _():
                    swait(); swait()
                @pl.when(g + 1 < NG)
                def _():
                    gstart((g + 1) * GS, oth[0])
                    gstart((g + 1) * GS + 1, oth[1])

            def body(gg, _):
                group(2 * gg, A, B)
                group(2 * gg + 1, B, A)
                return 0
            lax.fori_loop(0, NG // 2, body, 0)
            swait(); swait()

        @pl.when(cid == 0)
        def _():
            run(featA)
        @pl.when(cid == 1)
        def _():
            run(featB)
        plsc.subcore_barrier()

        OC = 320
        rbase = sid * RPT
        def ocp(base, nrows):
            pltpu.sync_copy(acc.at[pl.ds(base, nrows)],
                            rows0.at[pl.ds(0, nrows)])
            pltpu.sync_copy(rows0.at[pl.ds(0, nrows)],
                            out.at[cid, pl.ds(base, nrows)])
        ocp(rbase, OC)
        @pl.when(sid < NS - 1)
        def _():
            ocp(rbase + OC, OC)
        @pl.when(sid == NS - 1)
        def _():
            ocp(rbase + OC, RPT_LAST - OC)

    return k


def _make_deg_pass():
    W = 16

    @functools.partial(
        pl.kernel,
        out_type=jax.ShapeDtypeStruct((NC, N, W), jnp.float32),
        mesh=_MESH,
        scratch_types=[
            pltpu.VMEM((CPT_D, CHD), jnp.int32),
            pltpu.VMEM((CHD, W), jnp.float32),
            pltpu.VMEM((CHD, W), jnp.float32),
            pltpu.VMEM((RPT, W), jnp.float32),
            pltpu.VMEM_SHARED((N_ACC, W), jnp.float32),
        ],
        compiler_params=pltpu.CompilerParams(use_tc_tiling_on_sc=False),
    )
    def k(dstr, out, didx, ones, zeros, obuf, acc):
        cid = lax.axis_index("c")
        sid = lax.axis_index("s")
        wid = sid * NC + cid
        pltpu.sync_copy(dstr.at[wid], didx)

        def fill(i, _):
            ones[i, pl.ds(0, 16)] = jnp.ones((16,), jnp.float32)
            zeros[i, pl.ds(0, 16)] = jnp.zeros((16,), jnp.float32)
            return 0
        lax.fori_loop(0, CHD, fill, 0)
        zbase = sid * ZPT
        def zcp(i, _):
            pltpu.sync_copy(zeros, acc.at[pl.ds(zbase + i * CHD, CHD)])
            return 0
        lax.fori_loop(0, ZPT // CHD, zcp, 0)
        plsc.subcore_barrier()

        def body(j, _):
            pltpu.sync_copy(ones, acc.at[didx.at[j]], add=True)
            return 0
        lax.fori_loop(0, CPT_D, body, 0)
        plsc.subcore_barrier()

        rbase = sid * RPT
        @pl.when(sid < NS - 1)
        def _():
            pltpu.sync_copy(acc.at[pl.ds(rbase, RPT)], obuf)
            pltpu.sync_copy(obuf, out.at[cid, pl.ds(rbase, RPT)])
        @pl.when(sid == NS - 1)
        def _():
            pltpu.sync_copy(acc.at[pl.ds(rbase, RPT_LAST)],
                            obuf.at[pl.ds(0, RPT_LAST)])
            pltpu.sync_copy(obuf.at[pl.ds(0, RPT_LAST)],
                            out.at[cid, pl.ds(rbase, RPT_LAST)])

    return k


_EDGE = _make_edge_pass()
_DEG = _make_deg_pass()


def _instnorm_relu(t):
    m = jnp.mean(t, axis=1, keepdims=True)
    v = jnp.mean((t - m) ** 2, axis=1, keepdims=True)
    return jnp.maximum((t - m) * lax.rsqrt(v + EPS), 0.0)


def _bn_relu(z, g, b):
    m = jnp.mean(z, axis=0, keepdims=True)
    v = jnp.mean((z - m) ** 2, axis=0, keepdims=True)
    return jnp.maximum((z - m) * lax.rsqrt(v + EPS) * g + b, 0.0)


def _tc1(degp, x, W1):
    def body(degp_ref, x_ref, w1_ref, xs1_ref, dinv_ref):
        deg = degp_ref[0, :, 0:1] + degp_ref[1, :, 0:1] + 1.0
        dinv = lax.rsqrt(deg)
        xw = jnp.dot(x_ref[...], w1_ref[...],
                     preferred_element_type=jnp.float32)
        xs1_ref[...] = xw * dinv
        dinv_ref[...] = dinv
    return pl.pallas_call(
        body,
        out_shape=(jax.ShapeDtypeStruct((N, 64), jnp.float32),
                   jax.ShapeDtypeStruct((N, 1), jnp.float32)),
    )(degp, x, W1)


def _tc2(p, xs1, dinv, b1, W2):
    def body(p_ref, xs1_ref, dinv_ref, b1_ref, w2_ref, xs2_ref):
        dinv = dinv_ref[...]
        agg = jnp.concatenate([p_ref[0], p_ref[1]], axis=1)
        t = dinv * (agg + xs1_ref[...]) + b1_ref[...]
        h = _instnorm_relu(t)
        xw2 = jnp.dot(h, w2_ref[...], preferred_element_type=jnp.float32)
        xs2_ref[...] = xw2 * dinv
    return pl.pallas_call(
        body,
        out_shape=jax.ShapeDtypeStruct((N, 64), jnp.float32),
    )(p, xs1, dinv, b1, W2)


def _tc3(p, xs2, dinv, b2):
    def body(p_ref, xs2_ref, dinv_ref, b2_ref, h2_ref):
        agg = jnp.concatenate([p_ref[0], p_ref[1]], axis=1)
        t = dinv_ref[...] * (agg + xs2_ref[...]) + b2_ref[...]
        h2_ref[...] = _instnorm_relu(t)
    return pl.pallas_call(
        body,
        out_shape=jax.ShapeDtypeStruct((N, 64), jnp.float32),
    )(p, xs2, dinv, b2)


def _tc4(p, h2, g1W1, g1b1, g1W2, g1b2, bn1_g, bn1_b):
    def body(p_ref, h2_ref, wa_ref, ba_ref, wb_ref, bb_ref, g_ref, be_ref,
             a_ref):
        ain = h2_ref[...] + jnp.concatenate([p_ref[0], p_ref[1]], axis=1)
        z = jnp.maximum(jnp.dot(ain, wa_ref[...],
                                preferred_element_type=jnp.float32)
                        + ba_ref[...], 0.0)
        z = jnp.dot(z, wb_ref[...],
                    preferred_element_type=jnp.float32) + bb_ref[...]
        a_ref[...] = _bn_relu(z, g_ref[...], be_ref[...])
    return pl.pallas_call(
        body,
        out_shape=jax.ShapeDtypeStruct((N, 64), jnp.float32),
    )(p, h2, g1W1, g1b1, g1W2, g1b2, bn1_g, bn1_b)


def _tc5(p, a, g2W, g2b, bn2_g, bn2_b, batch, fcWt, fcb):
    def body(p_ref, a_ref, w_ref, b_ref, g_ref, be_ref, batch_ref, fcw_ref,
             fcb_ref, out_ref):
        a2 = a_ref[...] + jnp.concatenate([p_ref[0], p_ref[1]], axis=1)
        y = jnp.dot(a2, w_ref[...],
                    preferred_element_type=jnp.float32) + b_ref[...]
        y = _bn_relu(y, g_ref[...], be_ref[...])
        gid = lax.broadcasted_iota(jnp.int32, (G, N), 0)
        oh = (gid == jnp.broadcast_to(batch_ref[...], (G, N))
              ).astype(jnp.float32)
        pooled = jnp.dot(oh, y, preferred_element_type=jnp.float32)
        out_ref[...] = (jnp.sum(pooled * fcw_ref[...], axis=1, keepdims=True)
                        + fcb_ref[...])
    return pl.pallas_call(
        body,
        out_shape=jax.ShapeDtypeStruct((G, 1), jnp.float32),
    )(p, a, g2W, g2b, bn2_g, bn2_b, batch, fcWt, fcb)


def kernel(x, edge_index, batch, W1, b1, W2, b2, g1W1, g1b1, g1W2, g1b2,
           g2W, g2b, bn1_g, bn1_b, bn2_g, bn2_b, fcW, fcb):
    src = edge_index[0].astype(jnp.int32)
    dst = edge_index[1].astype(jnp.int32)
    pad = EP - E
    srcf = jnp.concatenate([src, jnp.zeros((pad,), jnp.int32)])
    dstf = jnp.concatenate([dst, jnp.full((pad,), DUMMY, jnp.int32)])
    srcp = srcf.reshape(NS, CPT, CH)
    dstp = dstf.reshape(NS, CPT, CH)
    dstd = dstf.reshape(NW, CPT_D, CHD)

    def halves(f):
        return f[:, :HW], f[:, HW:]

    degp = _DEG(dstd)
    xs1, dinv = _tc1(degp, x, W1)
    p1 = _EDGE(*halves(xs1), srcp, dstp)
    xs2 = _tc2(p1, xs1, dinv, b1.reshape(1, -1), W2)
    p2 = _EDGE(*halves(xs2), srcp, dstp)
    h2 = _tc3(p2, xs2, dinv, b2.reshape(1, -1))
    p3 = _EDGE(*halves(h2), srcp, dstp)
    a = _tc4(p3, h2, g1W1, g1b1.reshape(1, -1), g1W2, g1b2.reshape(1, -1),
             bn1_g.reshape(1, -1), bn1_b.reshape(1, -1))
    p4 = _EDGE(*halves(a), srcp, dstp)
    out = _tc5(p4, a, g2W, g2b.reshape(1, -1), bn2_g.reshape(1, -1),
               bn2_b.reshape(1, -1), batch.reshape(1, -1).astype(jnp.int32),
               fcW.reshape(1, -1), fcb.reshape(1, 1))
    return out


# speedup vs baseline: 1.0397x; 1.0397x over previous
import functools

import jax
import jax.numpy as jnp
from jax import lax
from jax.experimental import pallas as pl
from jax.experimental.pallas import tpu as pltpu
from jax.experimental.pallas import tpu_sc as plsc

NC, NS = 2, 16
NW = NC * NS
N = 10000
E = 320000
G = 16
EPS = 1e-5

CH = 1024
EP = 327680
EPT = EP // NS
CPT = EPT // CH
CHD = 128
EPT_D = EP // NW
CPT_D = EPT_D // CHD
N_ACC = 10240
DUMMY = N
ZPT = N_ACC // NS
RPT = 640
RPT_LAST = N - RPT * (NS - 1)
HW = 32

_MESH = plsc.VectorSubcoreMesh(core_axis_name="c", subcore_axis_name="s",
                               num_cores=NC, num_subcores=NS)


def _make_edge_pass():
    @functools.partial(
        pl.kernel,
        out_type=jax.ShapeDtypeStruct((NC, N, HW), jnp.float32),
        mesh=_MESH,
        scratch_types=[
            pltpu.VMEM((CPT, CH), jnp.int32),
            pltpu.VMEM((CPT, CH), jnp.int32),
            pltpu.VMEM((CH, HW), jnp.float32),
            pltpu.VMEM((CH, HW), jnp.float32),
            pltpu.VMEM_SHARED((N_ACC, HW), jnp.float32),
            pltpu.SemaphoreType.DMA,
            pltpu.SemaphoreType.DMA,
        ],
        compiler_params=pltpu.CompilerParams(use_tc_tiling_on_sc=False),
    )
    def k(featA, featB, srcr, dstr, out, sidx, didx, rows0, rows1, acc,
          gsem, ssem):
        cid = lax.axis_index("c")
        sid = lax.axis_index("s")
        pltpu.sync_copy(srcr.at[sid], sidx)
        pltpu.sync_copy(dstr.at[sid], didx)

        def zrow(i, _):
            for j in range(HW // 16):
                rows0[i, pl.ds(j * 16, 16)] = jnp.zeros((16,), jnp.float32)
            return 0
        lax.fori_loop(0, ZPT, zrow, 0)
        zbase = sid * ZPT
        pltpu.sync_copy(rows0.at[pl.ds(0, ZPT)], acc.at[pl.ds(zbase, ZPT)])
        plsc.subcore_barrier()

        def run(feat):
            def gstart(j, buf):
                pltpu.async_copy(feat.at[sidx.at[j]], buf, gsem)

            def gwait():
                pltpu.make_async_copy(feat.at[sidx.at[0]], rows0, gsem).wait()

            def sstart(j, buf):
                pltpu.async_copy(buf, acc.at[didx.at[j]], ssem, add=True)

            def swait():
                pltpu.make_async_copy(feat.at[pl.ds(0, CH)], rows0,
                                      ssem).wait()

            gstart(0, rows0)

            def group(g, cur, oth):
                gwait()
                sstart(g, cur)
                @pl.when(g > 0)
                def _():
                    swait()
                @pl.when(g + 1 < CPT)
                def _():
                    gstart(g + 1, oth)

            def body(gg, _):
                group(2 * gg, rows0, rows1)
                group(2 * gg + 1, rows1, rows0)
                return 0
            lax.fori_loop(0, CPT // 2, body, 0)
            swait()

        @pl.when(cid == 0)
        def _():
            run(featA)
        @pl.when(cid == 1)
        def _():
            run(featB)
        plsc.subcore_barrier()

        rbase = sid * RPT
        def ocp(base, nrows):
            pltpu.sync_copy(acc.at[pl.ds(base, nrows)],
                            rows0.at[pl.ds(0, nrows)])
            pltpu.sync_copy(rows0.at[pl.ds(0, nrows)],
                            out.at[cid, pl.ds(base, nrows)])
        @pl.when(sid < NS - 1)
        def _():
            ocp(rbase, RPT)
        @pl.when(sid == NS - 1)
        def _():
            ocp(rbase, RPT_LAST)

    return k


def _make_deg_pass():
    W = 16

    @functools.partial(
        pl.kernel,
        out_type=jax.ShapeDtypeStruct((NC, N, W), jnp.float32),
        mesh=_MESH,
        scratch_types=[
            pltpu.VMEM((CPT_D, CHD), jnp.int32),
            pltpu.VMEM((CHD, W), jnp.float32),
            pltpu.VMEM((CHD, W), jnp.float32),
            pltpu.VMEM((RPT, W), jnp.float32),
            pltpu.VMEM_SHARED((N_ACC, W), jnp.float32),
        ],
        compiler_params=pltpu.CompilerParams(use_tc_tiling_on_sc=False),
    )
    def k(dstr, out, didx, ones, zeros, obuf, acc):
        cid = lax.axis_index("c")
        sid = lax.axis_index("s")
        wid = sid * NC + cid
        pltpu.sync_copy(dstr.at[wid], didx)

        def fill(i, _):
            ones[i, pl.ds(0, 16)] = jnp.ones((16,), jnp.float32)
            zeros[i, pl.ds(0, 16)] = jnp.zeros((16,), jnp.float32)
            return 0
        lax.fori_loop(0, CHD, fill, 0)
        zbase = sid * ZPT
        def zcp(i, _):
            pltpu.sync_copy(zeros, acc.at[pl.ds(zbase + i * CHD, CHD)])
            return 0
        lax.fori_loop(0, ZPT // CHD, zcp, 0)
        plsc.subcore_barrier()

        def body(j, _):
            pltpu.sync_copy(ones, acc.at[didx.at[j]], add=True)
            return 0
        lax.fori_loop(0, CPT_D, body, 0)
        plsc.subcore_barrier()

        rbase = sid * RPT
        @pl.when(sid < NS - 1)
        def _():
            pltpu.sync_copy(acc.at[pl.ds(rbase, RPT)], obuf)
            pltpu.sync_copy(obuf, out.at[cid, pl.ds(rbase, RPT)])
        @pl.when(sid == NS - 1)
        def _():
            pltpu.sync_copy(acc.at[pl.ds(rbase, RPT_LAST)],
                            obuf.at[pl.ds(0, RPT_LAST)])
            pltpu.sync_copy(obuf.at[pl.ds(0, RPT_LAST)],
                            out.at[cid, pl.ds(rbase, RPT_LAST)])

    return k


_EDGE = _make_edge_pass()
_DEG = _make_deg_pass()


def _instnorm_relu(t):
    m = jnp.mean(t, axis=1, keepdims=True)
    v = jnp.mean((t - m) ** 2, axis=1, keepdims=True)
    return jnp.maximum((t - m) * lax.rsqrt(v + EPS), 0.0)


def _bn_relu(z, g, b):
    m = jnp.mean(z, axis=0, keepdims=True)
    v = jnp.mean((z - m) ** 2, axis=0, keepdims=True)
    return jnp.maximum((z - m) * lax.rsqrt(v + EPS) * g + b, 0.0)


def _tc1(degp, x, W1):
    def body(degp_ref, x_ref, w1_ref, xsA_ref, xsB_ref, dinv_ref):
        deg = degp_ref[0, :, 0:1] + degp_ref[1, :, 0:1] + 1.0
        dinv = lax.rsqrt(deg)
        xw = jnp.dot(x_ref[...], w1_ref[...],
                     preferred_element_type=jnp.float32)
        xs = xw * dinv
        xsA_ref[...] = xs[:, :HW]
        xsB_ref[...] = xs[:, HW:]
        dinv_ref[...] = dinv
    return pl.pallas_call(
        body,
        out_shape=(jax.ShapeDtypeStruct((N, HW), jnp.float32),
                   jax.ShapeDtypeStruct((N, HW), jnp.float32),
                   jax.ShapeDtypeStruct((N, 1), jnp.float32)),
    )(degp, x, W1)


def _tc2(p, xsA, xsB, dinv, b1, W2):
    def body(p_ref, xsA_ref, xsB_ref, dinv_ref, b1_ref, w2_ref,
             oA_ref, oB_ref):
        dinv = dinv_ref[...]
        t = jnp.concatenate([p_ref[0] + xsA_ref[...],
                             p_ref[1] + xsB_ref[...]], axis=1)
        t = dinv * t + b1_ref[...]
        h = _instnorm_relu(t)
        xs = jnp.dot(h, w2_ref[...], preferred_element_type=jnp.float32) * dinv
        oA_ref[...] = xs[:, :HW]
        oB_ref[...] = xs[:, HW:]
    return pl.pallas_call(
        body,
        out_shape=(jax.ShapeDtypeStruct((N, HW), jnp.float32),
                   jax.ShapeDtypeStruct((N, HW), jnp.float32)),
    )(p, xsA, xsB, dinv, b1, W2)


def _tc3(p, xsA, xsB, dinv, b2):
    def body(p_ref, xsA_ref, xsB_ref, dinv_ref, b2_ref, oA_ref, oB_ref):
        t = jnp.concatenate([p_ref[0] + xsA_ref[...],
                             p_ref[1] + xsB_ref[...]], axis=1)
        t = dinv_ref[...] * t + b2_ref[...]
        h2 = _instnorm_relu(t)
        oA_ref[...] = h2[:, :HW]
        oB_ref[...] = h2[:, HW:]
    return pl.pallas_call(
        body,
        out_shape=(jax.ShapeDtypeStruct((N, HW), jnp.float32),
                   jax.ShapeDtypeStruct((N, HW), jnp.float32)),
    )(p, xsA, xsB, dinv, b2)


def _tc4(p, h2A, h2B, g1W1, g1b1, g1W2, g1b2, bn1_g, bn1_b):
    def body(p_ref, h2A_ref, h2B_ref, wa_ref, ba_ref, wb_ref, bb_ref,
             g_ref, be_ref, oA_ref, oB_ref):
        ain = jnp.concatenate([h2A_ref[...] + p_ref[0],
                               h2B_ref[...] + p_ref[1]], axis=1)
        z = jnp.maximum(jnp.dot(ain, wa_ref[...],
                                preferred_element_type=jnp.float32)
                        + ba_ref[...], 0.0)
        z = jnp.dot(z, wb_ref[...],
                    preferred_element_type=jnp.float32) + bb_ref[...]
        za = _bn_relu(z, g_ref[...], be_ref[...])
        oA_ref[...] = za[:, :HW]
        oB_ref[...] = za[:, HW:]
    return pl.pallas_call(
        body,
        out_shape=(jax.ShapeDtypeStruct((N, HW), jnp.float32),
                   jax.ShapeDtypeStruct((N, HW), jnp.float32)),
    )(p, h2A, h2B, g1W1, g1b1, g1W2, g1b2, bn1_g, bn1_b)


def _tc5(p, aA, aB, g2W, g2b, bn2_g, bn2_b, batch, fcWt, fcb):
    def body(p_ref, aA_ref, aB_ref, w_ref, b_ref, g_ref, be_ref, batch_ref,
             fcw_ref, fcb_ref, out_ref):
        a2 = jnp.concatenate([aA_ref[...] + p_ref[0],
                              aB_ref[...] + p_ref[1]], axis=1)
        y = jnp.dot(a2, w_ref[...],
                    preferred_element_type=jnp.float32) + b_ref[...]
        y = _bn_relu(y, g_ref[...], be_ref[...])
        gid = lax.broadcasted_iota(jnp.int32, (G, N), 0)
        oh = (gid == jnp.broadcast_to(batch_ref[...], (G, N))
              ).astype(jnp.float32)
        pooled = jnp.dot(oh, y, preferred_element_type=jnp.float32)
        out_ref[...] = (jnp.sum(pooled * fcw_ref[...], axis=1, keepdims=True)
                        + fcb_ref[...])
    return pl.pallas_call(
        body,
        out_shape=jax.ShapeDtypeStruct((G, 1), jnp.float32),
    )(p, aA, aB, g2W, g2b, bn2_g, bn2_b, batch, fcWt, fcb)


def kernel(x, edge_index, batch, W1, b1, W2, b2, g1W1, g1b1, g1W2, g1b2,
           g2W, g2b, bn1_g, bn1_b, bn2_g, bn2_b, fcW, fcb):
    src = edge_index[0].astype(jnp.int32)
    dst = edge_index[1].astype(jnp.int32)
    pad = EP - E
    srcf = jnp.concatenate([src, jnp.zeros((pad,), jnp.int32)])
    dstf = jnp.concatenate([dst, jnp.full((pad,), DUMMY, jnp.int32)])
    srcp = srcf.reshape(NS, CPT, CH)
    dstp = dstf.reshape(NS, CPT, CH)
    dstd = dstf.reshape(NW, CPT_D, CHD)

    degp = _DEG(dstd)
    xs1A, xs1B, dinv = _tc1(degp, x, W1)
    p1 = _EDGE(xs1A, xs1B, srcp, dstp)
    xs2A, xs2B = _tc2(p1, xs1A, xs1B, dinv, b1.reshape(1, -1), W2)
    p2 = _EDGE(xs2A, xs2B, srcp, dstp)
    h2A, h2B = _tc3(p2, xs2A, xs2B, dinv, b2.reshape(1, -1))
    p3 = _EDGE(h2A, h2B, srcp, dstp)
    aA, aB = _tc4(p3, h2A, h2B, g1W1, g1b1.reshape(1, -1), g1W2,
                  g1b2.reshape(1, -1), bn1_g.reshape(1, -1),
                  bn1_b.reshape(1, -1))
    p4 = _EDGE(aA, aB, srcp, dstp)
    out = _tc5(p4, aA, aB, g2W, g2b.reshape(1, -1), bn2_g.reshape(1, -1),
               bn2_b.reshape(1, -1), batch.reshape(1, -1).astype(jnp.int32),
               fcW.reshape(1, -1), fcb.reshape(1, 1))
    return out
